# baseline (speedup 1.0000x reference)
import functools

import jax
import jax.numpy as jnp
from jax import lax
from jax.experimental import pallas as pl
from jax.experimental.pallas import tpu as pltpu
from jax.experimental.pallas import tpu_sc as plsc

ED = 768
CD = 256
OUT = 800
K = 8192
NH = 12
HD = ED // NH
MLP = 4 * ED
B = 16
N = 256
IN = 800
M = B * N
BM = 256
NRB = M // BM
BK = 512
NKB = K // BK

_f32 = jnp.float32


def _ln_block(a, g, b):
    m = jnp.mean(a, axis=1, keepdims=True)
    v = jnp.mean((a - m) ** 2, axis=1, keepdims=True)
    return g * (a - m) / jnp.sqrt(v + 1e-5) + b


_bf16 = jnp.bfloat16


def _dot(a, b):
    return jnp.dot(a.astype(_bf16), b, preferred_element_type=_f32)


def _block_body(x, refs, pre_n):
    (in_W, in_b, ln1_g, ln1_b, qkv_W, qkv_b, o_W, o_b,
     ln2_g, ln2_b, fc1_W, fc1_b, fc2_W, fc2_b) = [r[...] for r in refs]
    h = _dot(x, in_W) + in_b
    qkv = _dot(_ln_block(h, ln1_g, ln1_b), qkv_W) + qkv_b
    scale = 1.0 / (HD ** 0.5)
    parts = []
    qkv16 = qkv.astype(_bf16)
    for hh in range(NH):
        q = qkv16[:, hh * HD:(hh + 1) * HD]
        k = qkv16[:, ED + hh * HD:ED + (hh + 1) * HD]
        v = qkv16[:, 2 * ED + hh * HD:2 * ED + (hh + 1) * HD]
        s = lax.dot_general(q, k, (((1,), (1,)), ((), ())),
                            preferred_element_type=_f32) * scale
        mx = jnp.max(s, axis=1, keepdims=True)
        e = jnp.exp(s - mx)
        r = 1.0 / jnp.sum(e, axis=1, keepdims=True)
        parts.append(_dot(e, v) * r)
    o = jnp.concatenate(parts, axis=1)
    h = h + _dot(o, o_W) + o_b
    g = jax.nn.gelu((_dot(_ln_block(h, ln2_g, ln2_b), fc1_W)
                     + fc1_b).astype(_bf16))
    return h + _dot(g, fc2_W) + fc2_b


def _block_weights(P, pre):
    names = ["in_W", "in_b", "ln1_g", "ln1_b", "qkv_W", "qkv_b", "o_W",
             "o_b", "ln2_g", "ln2_b", "fc1_W", "fc1_b", "fc2_W", "fc2_b"]
    ws = []
    for n in names:
        w = P[pre + n]
        ws.append(w.reshape(1, -1) if w.ndim == 1 else w.astype(_bf16))
    return ws


def _const_specs(arrs):
    return [pl.BlockSpec(a.shape, lambda i, nd=a.ndim: (0,) * nd)
            for a in arrs]


def _enc_mega(x2d, P):
    ws = _block_weights(P, "enc_") + [
        P["ep1_W"], P["ep1_b"].reshape(1, ED),
        P["ep2_W"], P["ep2_b"].reshape(1, CD)]

    def body(x_ref, *refs):
        o_ref = refs[-1]
        h = _block_body(x_ref[...], refs[:14], "enc_")
        ep1_W, ep1_b, ep2_W, ep2_b = [r[...] for r in refs[14:18]]
        t = jnp.tanh((_dot(h, ep1_W) + ep1_b).astype(_bf16))
        z = _dot(t, ep2_W) + ep2_b
        n = jnp.sqrt(jnp.sum(z * z, axis=1, keepdims=True))
        o_ref[...] = z / jnp.maximum(n, 1e-12)

    return pl.pallas_call(
        body,
        grid=(NRB,),
        in_specs=[pl.BlockSpec((BM, IN), lambda i: (i, 0))] + _const_specs(ws),
        out_specs=pl.BlockSpec((BM, CD), lambda i: (i, 0)),
        out_shape=jax.ShapeDtypeStruct((M, CD), _f32),
    )(x2d.astype(_bf16), *ws)


def _dec_mega(zq, P, x2d, zn):
    ws = _block_weights(P, "dec_") + [
        P["dp1_W"], P["dp1_b"].reshape(1, ED),
        P["dp2_W"], P["dp2_b"].reshape(1, OUT)]

    def body(z_ref, *refs):
        x_ref, zn_ref, rec_ref, vq_ref = refs[-4], refs[-3], refs[-2], refs[-1]

        @pl.when(pl.program_id(0) == 0)
        def _init():
            rec_ref[...] = jnp.zeros((1, 1), _f32)
            vq_ref[...] = jnp.zeros((1, 1), _f32)

        zq_blk = z_ref[...]
        n = jnp.sqrt(jnp.sum(zq_blk * zq_blk, axis=1, keepdims=True))
        vn = zq_blk / jnp.maximum(n, 1e-12)
        dv = zn_ref[...] - vn
        vq_ref[...] += jnp.sum(dv * dv).reshape(1, 1)

        h = _block_body(zq_blk, refs[:14], "dec_")
        dp1_W, dp1_b, dp2_W, dp2_b = [r[...] for r in refs[14:18]]
        t = jnp.tanh((_dot(h, dp1_W) + dp1_b).astype(_bf16))
        xr = _dot(t, dp2_W) + dp2_b
        d = xr - x_ref[...]
        rec_ref[...] += jnp.sum(d * d).reshape(1, 1)

    return pl.pallas_call(
        body,
        grid=(NRB,),
        in_specs=([pl.BlockSpec((BM, CD), lambda i: (i, 0))]
                  + _const_specs(ws)
                  + [pl.BlockSpec((BM, OUT), lambda i: (i, 0)),
                     pl.BlockSpec((BM, CD), lambda i: (i, 0))]),
        out_specs=[pl.BlockSpec((1, 1), lambda i: (0, 0)),
                   pl.BlockSpec((1, 1), lambda i: (0, 0))],
        out_shape=[jax.ShapeDtypeStruct((1, 1), _f32),
                   jax.ShapeDtypeStruct((1, 1), _f32)],
    )(zq, *ws, x2d, zn)


def _vq_argmin(zn_bf16, Et_bf16):
    def body(zn_ref, et_ref, idx_ref):
        zn = zn_ref[...]
        bv = jnp.full((BM, 1), jnp.inf, _f32)
        bi = jnp.zeros((BM, 1), _f32)
        iota = lax.broadcasted_iota(jnp.int32, (BM, BK), 1).astype(_f32)
        for c in range(NKB):
            et = et_ref[:, c * BK:(c + 1) * BK]
            etf = et.astype(_f32)
            esq = jnp.sum(etf * etf, axis=0, keepdims=True)
            scores = jnp.dot(zn, et, preferred_element_type=_f32)
            val = esq - 2.0 * scores
            mn = jnp.min(val, axis=1, keepdims=True)
            am = jnp.min(jnp.where(val == mn, iota, float(K)), axis=1,
                         keepdims=True)
            gidx = am + float(BK) * c
            better = mn < bv
            bi = jnp.where(better, gidx, bi)
            bv = jnp.where(better, mn, bv)
        idx_ref[...] = bi

    blk = pl.BlockSpec
    return pl.pallas_call(
        body,
        grid=(NRB,),
        in_specs=[
            blk((BM, CD), lambda i: (i, 0)),
            blk((CD, K), lambda i: (0, 0)),
        ],
        out_specs=blk((BM, 1), lambda i: (i, 0)),
        out_shape=jax.ShapeDtypeStruct((M, 1), _f32),
    )(zn_bf16, Et_bf16)


def _sc_gather(table, idx):
    info = plsc.get_sparse_core_info()
    nw = info.num_cores * info.num_subcores
    b_per_w = M // nw
    mesh = plsc.VectorSubcoreMesh(core_axis_name="c", subcore_axis_name="s")

    @functools.partial(
        pl.kernel,
        mesh=mesh,
        out_type=jax.ShapeDtypeStruct((M, CD), _f32),
        scratch_types=[
            pltpu.VMEM((b_per_w,), jnp.int32),
            pltpu.VMEM((b_per_w, CD), _f32),
            pltpu.SemaphoreType.DMA,
        ],
    )
    def gather_kernel(table_hbm, idx_hbm, out_hbm, idx_v, rows_v, sem):
        wid = lax.axis_index("s") * info.num_cores + lax.axis_index("c")
        base = wid * b_per_w
        pltpu.sync_copy(idx_hbm.at[pl.ds(base, b_per_w)], idx_v)
        pltpu.async_copy(table_hbm.at[idx_v], rows_v, sem).wait()
        pltpu.sync_copy(rows_v, out_hbm.at[pl.ds(base, b_per_w)])

    return gather_kernel(table, idx)


def kernel(x, params):
    P = params
    x2d = x.reshape(M, IN)
    zn = _enc_mega(x2d, P)
    et16 = P["emb"].T.astype(jnp.bfloat16)
    idx = _vq_argmin(zn.astype(jnp.bfloat16), et16).reshape(M)
    zq = _sc_gather(P["emb"], idx.astype(jnp.int32))
    rec_sum, vq_sum = _dec_mega(zq, P, x2d, zn)
    return (rec_sum[0, 0] / (M * IN)
            + 1.25 * (vq_sum[0, 0] / (M * CD)))


# speedup vs baseline: 1.0321x; 1.0321x over previous
import functools

import jax
import jax.numpy as jnp
from jax import lax
from jax.experimental import pallas as pl
from jax.experimental.pallas import tpu as pltpu
from jax.experimental.pallas import tpu_sc as plsc

ED = 768
CD = 256
OUT = 800
K = 8192
NH = 12
HD = ED // NH
MLP = 4 * ED
B = 16
N = 256
IN = 800
M = B * N
BM = 256
NRB = M // BM
BK = 512
NKB = K // BK

_f32 = jnp.float32


def _ln_block(a, g, b):
    m = jnp.mean(a, axis=1, keepdims=True)
    v = jnp.mean((a - m) ** 2, axis=1, keepdims=True)
    return g * (a - m) / jnp.sqrt(v + 1e-5) + b


_bf16 = jnp.bfloat16


def _dot(a, b):
    return jnp.dot(a.astype(_bf16), b, preferred_element_type=_f32)


def _block_body(x, refs, pre_n):
    (in_W, in_b, ln1_g, ln1_b, qkv_W, qkv_b, o_W, o_b,
     ln2_g, ln2_b, fc1_W, fc1_b, fc2_W, fc2_b) = [r[...] for r in refs]
    h = _dot(x, in_W) + in_b
    qkv = _dot(_ln_block(h, ln1_g, ln1_b), qkv_W) + qkv_b
    scale = 1.0 / (HD ** 0.5)
    parts = []
    qkv16 = qkv.astype(_bf16)
    for hh in range(NH):
        q = qkv16[:, hh * HD:(hh + 1) * HD]
        k = qkv16[:, ED + hh * HD:ED + (hh + 1) * HD]
        v = qkv16[:, 2 * ED + hh * HD:2 * ED + (hh + 1) * HD]
        s = lax.dot_general(q, k, (((1,), (1,)), ((), ())),
                            preferred_element_type=_f32) * scale
        mx = jnp.max(s, axis=1, keepdims=True)
        e = jnp.exp(s - mx)
        r = 1.0 / jnp.sum(e, axis=1, keepdims=True)
        parts.append(_dot(e, v) * r)
    o = jnp.concatenate(parts, axis=1)
    h = h + _dot(o, o_W) + o_b
    g = jax.nn.gelu((_dot(_ln_block(h, ln2_g, ln2_b), fc1_W)
                     + fc1_b).astype(_bf16))
    return h + _dot(g, fc2_W) + fc2_b


def _block_weights(P, pre):
    names = ["in_W", "in_b", "ln1_g", "ln1_b", "qkv_W", "qkv_b", "o_W",
             "o_b", "ln2_g", "ln2_b", "fc1_W", "fc1_b", "fc2_W", "fc2_b"]
    ws = []
    for n in names:
        w = P[pre + n]
        ws.append(w.reshape(1, -1) if w.ndim == 1 else w.astype(_bf16))
    return ws


def _const_specs(arrs):
    return [pl.BlockSpec(a.shape, lambda i, nd=a.ndim: (0,) * nd)
            for a in arrs]


def _enc_mega(x2d, P):
    ws = _block_weights(P, "enc_") + [
        P["ep1_W"], P["ep1_b"].reshape(1, ED),
        P["ep2_W"], P["ep2_b"].reshape(1, CD)]

    def body(x_ref, *refs):
        et_ref, zn_ref, idx_ref = refs[-3], refs[-2], refs[-1]
        h = _block_body(x_ref[...], refs[:14], "enc_")
        ep1_W, ep1_b, ep2_W, ep2_b = [r[...] for r in refs[14:18]]
        t = jnp.tanh((_dot(h, ep1_W) + ep1_b).astype(_bf16))
        z = _dot(t, ep2_W) + ep2_b
        n = jnp.sqrt(jnp.sum(z * z, axis=1, keepdims=True))
        zn = z / jnp.maximum(n, 1e-12)
        zn_ref[...] = zn
        zn16 = zn.astype(_bf16)
        bv = jnp.full((BM, 1), jnp.inf, _f32)
        bi = jnp.zeros((BM, 1), _f32)
        iota = lax.broadcasted_iota(jnp.int32, (BM, BK), 1).astype(_f32)
        for c in range(NKB):
            et = et_ref[:, c * BK:(c + 1) * BK]
            etf = et.astype(_f32)
            esq = jnp.sum(etf * etf, axis=0, keepdims=True)
            scores = jnp.dot(zn16, et, preferred_element_type=_f32)
            val = esq - 2.0 * scores
            mn = jnp.min(val, axis=1, keepdims=True)
            am = jnp.min(jnp.where(val == mn, iota, float(K)), axis=1,
                         keepdims=True)
            gidx = am + float(BK) * c
            better = mn < bv
            bi = jnp.where(better, gidx, bi)
            bv = jnp.where(better, mn, bv)
        idx_ref[...] = bi.astype(jnp.int32)

    return pl.pallas_call(
        body,
        grid=(NRB,),
        in_specs=([pl.BlockSpec((BM, IN), lambda i: (i, 0))]
                  + _const_specs(ws)
                  + [pl.BlockSpec((CD, K), lambda i: (0, 0))]),
        out_specs=[pl.BlockSpec((BM, CD), lambda i: (i, 0)),
                   pl.BlockSpec((BM, 1), lambda i: (i, 0))],
        out_shape=[jax.ShapeDtypeStruct((M, CD), _f32),
                   jax.ShapeDtypeStruct((M, 1), jnp.int32)],
    )(x2d, *ws, P["emb"].T.astype(_bf16))


def _dec_mega(zq, P, x2d, zn):
    ws = _block_weights(P, "dec_") + [
        P["dp1_W"], P["dp1_b"].reshape(1, ED),
        P["dp2_W"], P["dp2_b"].reshape(1, OUT)]

    def body(z_ref, *refs):
        x_ref, zn_ref, rec_ref, vq_ref = refs[-4], refs[-3], refs[-2], refs[-1]

        @pl.when(pl.program_id(0) == 0)
        def _init():
            rec_ref[...] = jnp.zeros((1, 1), _f32)
            vq_ref[...] = jnp.zeros((1, 1), _f32)

        zq_blk = z_ref[...]
        n = jnp.sqrt(jnp.sum(zq_blk * zq_blk, axis=1, keepdims=True))
        vn = zq_blk / jnp.maximum(n, 1e-12)
        dv = zn_ref[...] - vn
        vq_ref[...] += jnp.sum(dv * dv).reshape(1, 1)

        h = _block_body(zq_blk, refs[:14], "dec_")
        dp1_W, dp1_b, dp2_W, dp2_b = [r[...] for r in refs[14:18]]
        t = jnp.tanh((_dot(h, dp1_W) + dp1_b).astype(_bf16))
        xr = _dot(t, dp2_W) + dp2_b
        d = xr - x_ref[...]
        rec_ref[...] += jnp.sum(d * d).reshape(1, 1)

    return pl.pallas_call(
        body,
        grid=(NRB,),
        in_specs=([pl.BlockSpec((BM, CD), lambda i: (i, 0))]
                  + _const_specs(ws)
                  + [pl.BlockSpec((BM, OUT), lambda i: (i, 0)),
                     pl.BlockSpec((BM, CD), lambda i: (i, 0))]),
        out_specs=[pl.BlockSpec((1, 1), lambda i: (0, 0)),
                   pl.BlockSpec((1, 1), lambda i: (0, 0))],
        out_shape=[jax.ShapeDtypeStruct((1, 1), _f32),
                   jax.ShapeDtypeStruct((1, 1), _f32)],
    )(zq, *ws, x2d, zn)


def _sc_gather(table, idx):
    info = plsc.get_sparse_core_info()
    nw = info.num_cores * info.num_subcores
    b_per_w = M // nw
    mesh = plsc.VectorSubcoreMesh(core_axis_name="c", subcore_axis_name="s")

    @functools.partial(
        pl.kernel,
        mesh=mesh,
        out_type=jax.ShapeDtypeStruct((M, CD), _f32),
        scratch_types=[
            pltpu.VMEM((b_per_w,), jnp.int32),
            pltpu.VMEM((b_per_w, CD), _f32),
            pltpu.SemaphoreType.DMA,
        ],
    )
    def gather_kernel(table_hbm, idx_hbm, out_hbm, idx_v, rows_v, sem):
        wid = lax.axis_index("s") * info.num_cores + lax.axis_index("c")
        base = wid * b_per_w
        pltpu.sync_copy(idx_hbm.at[pl.ds(base, b_per_w)], idx_v)
        pltpu.async_copy(table_hbm.at[idx_v], rows_v, sem).wait()
        pltpu.sync_copy(rows_v, out_hbm.at[pl.ds(base, b_per_w)])

    return gather_kernel(table, idx)


def kernel(x, params):
    P = params
    x2d = x.reshape(M, IN)
    zn, idx = _enc_mega(x2d, P)
    zq = _sc_gather(P["emb"], idx.reshape(M))
    rec_sum, vq_sum = _dec_mega(zq, P, x2d, zn)
    return (rec_sum[0, 0] / (M * IN)
            + 1.25 * (vq_sum[0, 0] / (M * CD)))


# speedup vs baseline: 1.0369x; 1.0047x over previous
import functools

import jax
import jax.numpy as jnp
from jax import lax
from jax.experimental import pallas as pl
from jax.experimental.pallas import tpu as pltpu
from jax.experimental.pallas import tpu_sc as plsc

ED = 768
CD = 256
OUT = 800
K = 8192
NH = 12
HD = ED // NH
MLP = 4 * ED
B = 16
N = 256
IN = 800
M = B * N
BM = 256
NRB = M // BM
BK = 512
NKB = K // BK

_f32 = jnp.float32


def _ln_block(a, g, b):
    m = jnp.mean(a, axis=1, keepdims=True)
    v = jnp.mean((a - m) ** 2, axis=1, keepdims=True)
    return g * (a - m) / jnp.sqrt(v + 1e-5) + b


_bf16 = jnp.bfloat16


def _dot(a, b):
    return jnp.dot(a.astype(_bf16), b, preferred_element_type=_f32)


def _block_body(x, refs, pre_n):
    (in_W, in_b, ln1_g, ln1_b, qkv_W, qkv_b, o_W, o_b,
     ln2_g, ln2_b, fc1_W, fc1_b, fc2_W, fc2_b) = [r[...] for r in refs]
    h = _dot(x, in_W) + in_b
    qkv = _dot(_ln_block(h, ln1_g, ln1_b), qkv_W) + qkv_b
    scale = 1.0 / (HD ** 0.5)
    parts = []
    qkv16 = qkv.astype(_bf16)
    for hh in range(NH):
        q = qkv16[:, hh * HD:(hh + 1) * HD]
        k = qkv16[:, ED + hh * HD:ED + (hh + 1) * HD]
        v = qkv16[:, 2 * ED + hh * HD:2 * ED + (hh + 1) * HD]
        s = lax.dot_general(q, k, (((1,), (1,)), ((), ())),
                            preferred_element_type=_f32) * scale
        mx = jnp.max(s, axis=1, keepdims=True)
        e = jnp.exp(s - mx)
        r = 1.0 / jnp.sum(e, axis=1, keepdims=True)
        parts.append(_dot(e, v) * r)
    o = jnp.concatenate(parts, axis=1)
    h = h + _dot(o, o_W) + o_b
    g = jax.nn.gelu((_dot(_ln_block(h, ln2_g, ln2_b), fc1_W)
                     + fc1_b).astype(_bf16))
    return h + _dot(g, fc2_W) + fc2_b


def _block_weights(P, pre):
    names = ["in_W", "in_b", "ln1_g", "ln1_b", "qkv_W", "qkv_b", "o_W",
             "o_b", "ln2_g", "ln2_b", "fc1_W", "fc1_b", "fc2_W", "fc2_b"]
    ws = []
    for n in names:
        w = P[pre + n]
        ws.append(w.reshape(1, -1) if w.ndim == 1 else w.astype(_bf16))
    return ws


def _const_specs(arrs):
    return [pl.BlockSpec(a.shape, lambda i, nd=a.ndim: (0,) * nd)
            for a in arrs]


def _enc_mega(x2d, P):
    ws = _block_weights(P, "enc_") + [
        P["ep1_W"], P["ep1_b"].reshape(1, ED),
        P["ep2_W"], P["ep2_b"].reshape(1, CD)]

    def body(x_ref, *refs):
        et_ref, zn_ref, idx_ref = refs[-3], refs[-2], refs[-1]
        h = _block_body(x_ref[...], refs[:14], "enc_")
        ep1_W, ep1_b, ep2_W, ep2_b = [r[...] for r in refs[14:18]]
        t = jnp.tanh((_dot(h, ep1_W) + ep1_b).astype(_bf16))
        z = _dot(t, ep2_W) + ep2_b
        n = jnp.sqrt(jnp.sum(z * z, axis=1, keepdims=True))
        zn = z / jnp.maximum(n, 1e-12)
        zn_ref[...] = zn
        zn16 = zn.astype(_bf16)
        bv = jnp.full((BM, 1), jnp.inf, _f32)
        bi = jnp.zeros((BM, 1), _f32)
        iota = lax.broadcasted_iota(jnp.int32, (BM, BK), 1).astype(_f32)
        ones_cd = jnp.ones((1, CD), _f32).astype(_bf16)
        for c in range(NKB):
            e = et_ref[c * BK:(c + 1) * BK, :]
            scores = lax.dot_general(zn16, e, (((1,), (1,)), ((), ())),
                                     preferred_element_type=_f32)
            esq = lax.dot_general(ones_cd, e * e, (((1,), (1,)), ((), ())),
                                  preferred_element_type=_f32)
            val = esq - 2.0 * scores
            mn = jnp.min(val, axis=1, keepdims=True)
            am = jnp.min(jnp.where(val == mn, iota, float(K)), axis=1,
                         keepdims=True)
            gidx = am + float(BK) * c
            better = mn < bv
            bi = jnp.where(better, gidx, bi)
            bv = jnp.where(better, mn, bv)
        idx_ref[...] = bi.astype(jnp.int32)

    return pl.pallas_call(
        body,
        grid=(NRB,),
        in_specs=([pl.BlockSpec((BM, IN), lambda i: (i, 0))]
                  + _const_specs(ws)
                  + [pl.BlockSpec((K, CD), lambda i: (0, 0))]),
        out_specs=[pl.BlockSpec((BM, CD), lambda i: (i, 0)),
                   pl.BlockSpec((BM, 1), lambda i: (i, 0))],
        out_shape=[jax.ShapeDtypeStruct((M, CD), _f32),
                   jax.ShapeDtypeStruct((M, 1), jnp.int32)],
    )(x2d, *ws, P["emb"].astype(_bf16))


def _dec_mega(zq, P, x2d, zn):
    ws = _block_weights(P, "dec_") + [
        P["dp1_W"], P["dp1_b"].reshape(1, ED),
        P["dp2_W"], P["dp2_b"].reshape(1, OUT)]

    def body(z_ref, *refs):
        x_ref, zn_ref, loss_ref = refs[-3], refs[-2], refs[-1]

        @pl.when(pl.program_id(0) == 0)
        def _init():
            loss_ref[...] = jnp.zeros((1, 1), _f32)

        zq_blk = z_ref[...]
        n = jnp.sqrt(jnp.sum(zq_blk * zq_blk, axis=1, keepdims=True))
        vn = zq_blk / jnp.maximum(n, 1e-12)
        dv = zn_ref[...] - vn
        vq_part = jnp.sum(dv * dv)

        h = _block_body(zq_blk, refs[:14], "dec_")
        dp1_W, dp1_b, dp2_W, dp2_b = [r[...] for r in refs[14:18]]
        t = jnp.tanh((_dot(h, dp1_W) + dp1_b).astype(_bf16))
        xr = _dot(t, dp2_W) + dp2_b
        d = xr - x_ref[...]
        rec_part = jnp.sum(d * d)
        loss_ref[...] += (rec_part * (1.0 / (M * IN))
                          + vq_part * (1.25 / (M * CD))).reshape(1, 1)

    return pl.pallas_call(
        body,
        grid=(NRB,),
        in_specs=([pl.BlockSpec((BM, CD), lambda i: (i, 0))]
                  + _const_specs(ws)
                  + [pl.BlockSpec((BM, OUT), lambda i: (i, 0)),
                     pl.BlockSpec((BM, CD), lambda i: (i, 0))]),
        out_specs=pl.BlockSpec((1, 1), lambda i: (0, 0)),
        out_shape=jax.ShapeDtypeStruct((1, 1), _f32),
    )(zq, *ws, x2d, zn)


def _sc_gather(table, idx):
    info = plsc.get_sparse_core_info()
    nw = info.num_cores * info.num_subcores
    b_per_w = M // nw
    mesh = plsc.VectorSubcoreMesh(core_axis_name="c", subcore_axis_name="s")

    @functools.partial(
        pl.kernel,
        mesh=mesh,
        out_type=jax.ShapeDtypeStruct((M, CD), _f32),
        scratch_types=[
            pltpu.VMEM((b_per_w,), jnp.int32),
            pltpu.VMEM((b_per_w, CD), _f32),
            pltpu.SemaphoreType.DMA,
        ],
    )
    def gather_kernel(table_hbm, idx_hbm, out_hbm, idx_v, rows_v, sem):
        wid = lax.axis_index("s") * info.num_cores + lax.axis_index("c")
        base = wid * b_per_w
        pltpu.sync_copy(idx_hbm.at[pl.ds(base, b_per_w)], idx_v)
        pltpu.async_copy(table_hbm.at[idx_v], rows_v, sem).wait()
        pltpu.sync_copy(rows_v, out_hbm.at[pl.ds(base, b_per_w)])

    return gather_kernel(table, idx)


def kernel(x, params):
    P = params
    x2d = x.reshape(M, IN)
    zn, idx = _enc_mega(x2d, P)
    zq = _sc_gather(P["emb"], idx.reshape(M))
    return _dec_mega(zq, P, x2d, zn)[0, 0]


# speedup vs baseline: 1.0594x; 1.0218x over previous
import functools

import jax
import jax.numpy as jnp
from jax import lax
from jax.experimental import pallas as pl
from jax.experimental.pallas import tpu as pltpu
from jax.experimental.pallas import tpu_sc as plsc

ED = 768
CD = 256
OUT = 800
K = 8192
NH = 12
HD = ED // NH
MLP = 4 * ED
B = 16
N = 256
IN = 800
M = B * N
BM = 256
NRB = M // BM
BK = 256
NKB = K // BK

_f32 = jnp.float32


def _ln_block(a, g, b):
    m = jnp.mean(a, axis=1, keepdims=True)
    v = jnp.mean((a - m) ** 2, axis=1, keepdims=True)
    return g * (a - m) / jnp.sqrt(v + 1e-5) + b


_bf16 = jnp.bfloat16


def _dot(a, b):
    return jnp.dot(a.astype(_bf16), b, preferred_element_type=_f32)


def _block_body(x, refs, pre_n):
    (in_W, in_b, ln1_g, ln1_b, qkv_W, qkv_b, o_W, o_b,
     ln2_g, ln2_b, fc1_W, fc1_b, fc2_W, fc2_b) = [r[...] for r in refs]
    h = _dot(x, in_W) + in_b
    qkv = _dot(_ln_block(h, ln1_g, ln1_b), qkv_W) + qkv_b
    scale = 1.0 / (HD ** 0.5)
    parts = []
    qkv16 = qkv.astype(_bf16)
    for hh in range(NH):
        q = qkv16[:, hh * HD:(hh + 1) * HD]
        k = qkv16[:, ED + hh * HD:ED + (hh + 1) * HD]
        v = qkv16[:, 2 * ED + hh * HD:2 * ED + (hh + 1) * HD]
        s = lax.dot_general(q, k, (((1,), (1,)), ((), ())),
                            preferred_element_type=_f32) * scale
        mx = jnp.max(s, axis=1, keepdims=True)
        e = jnp.exp(s - mx)
        r = 1.0 / jnp.sum(e, axis=1, keepdims=True)
        parts.append(_dot(e, v) * r)
    o = jnp.concatenate(parts, axis=1)
    h = h + _dot(o, o_W) + o_b
    g = jax.nn.gelu((_dot(_ln_block(h, ln2_g, ln2_b), fc1_W)
                     + fc1_b).astype(_bf16))
    return h + _dot(g, fc2_W) + fc2_b


def _block_weights(P, pre):
    names = ["in_W", "in_b", "ln1_g", "ln1_b", "qkv_W", "qkv_b", "o_W",
             "o_b", "ln2_g", "ln2_b", "fc1_W", "fc1_b", "fc2_W", "fc2_b"]
    ws = []
    for n in names:
        w = P[pre + n]
        ws.append(w.reshape(1, -1) if w.ndim == 1 else w.astype(_bf16))
    return ws


def _const_specs(arrs):
    return [pl.BlockSpec(a.shape, lambda i, nd=a.ndim: (0,) * nd)
            for a in arrs]


def _enc_mega(x2d, P):
    ws = _block_weights(P, "enc_") + [
        P["ep1_W"], P["ep1_b"].reshape(1, ED),
        P["ep2_W"], P["ep2_b"].reshape(1, CD)]

    def body(x_ref, *refs):
        et_ref, zn_ref, idx_ref = refs[-3], refs[-2], refs[-1]
        h = _block_body(x_ref[...], refs[:14], "enc_")
        ep1_W, ep1_b, ep2_W, ep2_b = [r[...] for r in refs[14:18]]
        t = jnp.tanh((_dot(h, ep1_W) + ep1_b).astype(_bf16))
        z = _dot(t, ep2_W) + ep2_b
        n = jnp.sqrt(jnp.sum(z * z, axis=1, keepdims=True))
        zn = z / jnp.maximum(n, 1e-12)
        zn_ref[...] = zn
        zn16 = zn.astype(_bf16)
        bv = jnp.full((BM, 1), jnp.inf, _f32)
        bi = jnp.zeros((BM, 1), _f32)
        iota = lax.broadcasted_iota(jnp.int32, (BM, BK), 1).astype(_f32)
        ones_cd = jnp.ones((1, CD), _f32).astype(_bf16)
        for c in range(NKB):
            e = et_ref[c * BK:(c + 1) * BK, :]
            scores = lax.dot_general(zn16, e, (((1,), (1,)), ((), ())),
                                     preferred_element_type=_f32)
            esq = lax.dot_general(ones_cd, e * e, (((1,), (1,)), ((), ())),
                                  preferred_element_type=_f32)
            val = esq - 2.0 * scores
            mn = jnp.min(val, axis=1, keepdims=True)
            am = jnp.min(jnp.where(val == mn, iota, float(K)), axis=1,
                         keepdims=True)
            gidx = am + float(BK) * c
            better = mn < bv
            bi = jnp.where(better, gidx, bi)
            bv = jnp.where(better, mn, bv)
        idx_ref[...] = bi.astype(jnp.int32)

    return pl.pallas_call(
        body,
        grid=(NRB,),
        in_specs=([pl.BlockSpec((BM, IN), lambda i: (i, 0))]
                  + _const_specs(ws)
                  + [pl.BlockSpec((K, CD), lambda i: (0, 0))]),
        out_specs=[pl.BlockSpec((BM, CD), lambda i: (i, 0)),
                   pl.BlockSpec((BM, 1), lambda i: (i, 0))],
        out_shape=[jax.ShapeDtypeStruct((M, CD), _f32),
                   jax.ShapeDtypeStruct((M, 1), jnp.int32)],
    )(x2d, *ws, P["emb"].astype(_bf16))


def _dec_mega(zq, P, x2d, zn):
    ws = _block_weights(P, "dec_") + [
        P["dp1_W"], P["dp1_b"].reshape(1, ED),
        P["dp2_W"], P["dp2_b"].reshape(1, OUT)]

    def body(z_ref, *refs):
        x_ref, zn_ref, loss_ref = refs[-3], refs[-2], refs[-1]

        @pl.when(pl.program_id(0) == 0)
        def _init():
            loss_ref[...] = jnp.zeros((1, 1), _f32)

        zq_blk = z_ref[...]
        n = jnp.sqrt(jnp.sum(zq_blk * zq_blk, axis=1, keepdims=True))
        vn = zq_blk / jnp.maximum(n, 1e-12)
        dv = zn_ref[...] - vn
        vq_part = jnp.sum(dv * dv)

        h = _block_body(zq_blk, refs[:14], "dec_")
        dp1_W, dp1_b, dp2_W, dp2_b = [r[...] for r in refs[14:18]]
        t = jnp.tanh((_dot(h, dp1_W) + dp1_b).astype(_bf16))
        xr = _dot(t, dp2_W) + dp2_b
        d = xr - x_ref[...]
        rec_part = jnp.sum(d * d)
        loss_ref[...] += (rec_part * (1.0 / (M * IN))
                          + vq_part * (1.25 / (M * CD))).reshape(1, 1)

    return pl.pallas_call(
        body,
        grid=(NRB,),
        in_specs=([pl.BlockSpec((BM, CD), lambda i: (i, 0))]
                  + _const_specs(ws)
                  + [pl.BlockSpec((BM, OUT), lambda i: (i, 0)),
                     pl.BlockSpec((BM, CD), lambda i: (i, 0))]),
        out_specs=pl.BlockSpec((1, 1), lambda i: (0, 0)),
        out_shape=jax.ShapeDtypeStruct((1, 1), _f32),
    )(zq, *ws, x2d, zn)


def _sc_gather(table, idx):
    info = plsc.get_sparse_core_info()
    nw = info.num_cores * info.num_subcores
    b_per_w = M // nw
    mesh = plsc.VectorSubcoreMesh(core_axis_name="c", subcore_axis_name="s")

    @functools.partial(
        pl.kernel,
        mesh=mesh,
        out_type=jax.ShapeDtypeStruct((M, CD), _f32),
        scratch_types=[
            pltpu.VMEM((b_per_w,), jnp.int32),
            pltpu.VMEM((b_per_w, CD), _f32),
            pltpu.SemaphoreType.DMA,
        ],
    )
    def gather_kernel(table_hbm, idx_hbm, out_hbm, idx_v, rows_v, sem):
        wid = lax.axis_index("s") * info.num_cores + lax.axis_index("c")
        base = wid * b_per_w
        pltpu.sync_copy(idx_hbm.at[pl.ds(base, b_per_w)], idx_v)
        pltpu.async_copy(table_hbm.at[idx_v], rows_v, sem).wait()
        pltpu.sync_copy(rows_v, out_hbm.at[pl.ds(base, b_per_w)])

    return gather_kernel(table, idx)


def kernel(x, params):
    P = params
    x2d = x.reshape(M, IN)
    zn, idx = _enc_mega(x2d, P)
    zq = _sc_gather(P["emb"], idx.reshape(M))
    return _dec_mega(zq, P, x2d, zn)[0, 0]


# speedup vs baseline: 1.0698x; 1.0098x over previous
import functools

import jax
import jax.numpy as jnp
from jax import lax
from jax.experimental import pallas as pl
from jax.experimental.pallas import tpu as pltpu
from jax.experimental.pallas import tpu_sc as plsc

ED = 768
CD = 256
OUT = 800
K = 8192
NH = 12
HD = ED // NH
MLP = 4 * ED
B = 16
N = 256
IN = 800
M = B * N
BM = 256
NRB = M // BM
BK = 256
NKB = K // BK

_f32 = jnp.float32


def _ln_block(a, g, b):
    m = jnp.mean(a, axis=1, keepdims=True)
    v = jnp.mean((a - m) ** 2, axis=1, keepdims=True)
    return g * (a - m) / jnp.sqrt(v + 1e-5) + b


_bf16 = jnp.bfloat16


def _dot(a, b):
    return jnp.dot(a.astype(_bf16), b, preferred_element_type=_f32)


def _block_body(x, refs, pre_n):
    (in_W, in_b, ln1_g, ln1_b, qkv_W, qkv_b, o_W, o_b,
     ln2_g, ln2_b, fc1_W, fc1_b, fc2_W, fc2_b) = [r[...] for r in refs]
    h = _dot(x, in_W) + in_b
    qkv = _dot(_ln_block(h, ln1_g, ln1_b), qkv_W) + qkv_b
    scale = 1.0 / (HD ** 0.5)
    parts = []
    qkv16 = qkv.astype(_bf16)
    for hh in range(NH):
        q = qkv16[:, hh * HD:(hh + 1) * HD]
        k = qkv16[:, ED + hh * HD:ED + (hh + 1) * HD]
        v = qkv16[:, 2 * ED + hh * HD:2 * ED + (hh + 1) * HD]
        s = lax.dot_general(q, k, (((1,), (1,)), ((), ())),
                            preferred_element_type=_f32) * scale
        mx = jnp.max(s, axis=1, keepdims=True)
        e = jnp.exp(s - mx)
        r = 1.0 / jnp.sum(e, axis=1, keepdims=True)
        parts.append(_dot(e, v) * r)
    o = jnp.concatenate(parts, axis=1)
    h = h + _dot(o, o_W) + o_b
    g = jax.nn.gelu((_dot(_ln_block(h, ln2_g, ln2_b), fc1_W)
                     + fc1_b).astype(_bf16))
    return h + _dot(g, fc2_W) + fc2_b


def _block_weights(P, pre):
    names = ["in_W", "in_b", "ln1_g", "ln1_b", "qkv_W", "qkv_b", "o_W",
             "o_b", "ln2_g", "ln2_b", "fc1_W", "fc1_b", "fc2_W", "fc2_b"]
    ws = []
    for n in names:
        w = P[pre + n]
        ws.append(w.reshape(1, -1) if w.ndim == 1 else w.astype(_bf16))
    return ws


def _const_specs(arrs):
    return [pl.BlockSpec(a.shape, lambda i, nd=a.ndim: (0,) * nd)
            for a in arrs]


def _enc_mega(x2d, P):
    ws = _block_weights(P, "enc_") + [
        P["ep1_W"], P["ep1_b"].reshape(1, ED),
        P["ep2_W"], P["ep2_b"].reshape(1, CD)]

    def body(x_ref, *refs):
        emb_ref, zn_ref, idx_ref, et_ref = refs[-4], refs[-3], refs[-2], refs[-1]

        @pl.when(pl.program_id(0) == 0)
        def _stage_codebook():
            for c in range(NKB):
                sl = pl.ds(c * BK, BK)
                et_ref[sl, :] = emb_ref[sl, :].astype(_bf16)

        h = _block_body(x_ref[...], refs[:14], "enc_")
        ep1_W, ep1_b, ep2_W, ep2_b = [r[...] for r in refs[14:18]]
        t = jnp.tanh((_dot(h, ep1_W) + ep1_b).astype(_bf16))
        z = _dot(t, ep2_W) + ep2_b
        n = jnp.sqrt(jnp.sum(z * z, axis=1, keepdims=True))
        zn = z / jnp.maximum(n, 1e-12)
        zn_ref[...] = zn
        zn16 = zn.astype(_bf16)
        bv = jnp.full((BM, 1), jnp.inf, _f32)
        bi = jnp.zeros((BM, 1), _f32)
        iota = lax.broadcasted_iota(jnp.int32, (BM, BK), 1).astype(_f32)
        ones_cd = jnp.ones((1, CD), _f32).astype(_bf16)
        for c in range(NKB):
            e = et_ref[c * BK:(c + 1) * BK, :]
            scores = lax.dot_general(zn16, e, (((1,), (1,)), ((), ())),
                                     preferred_element_type=_f32)
            esq = lax.dot_general(ones_cd, e * e, (((1,), (1,)), ((), ())),
                                  preferred_element_type=_f32)
            val = esq - 2.0 * scores
            mn = jnp.min(val, axis=1, keepdims=True)
            am = jnp.min(jnp.where(val == mn, iota, float(K)), axis=1,
                         keepdims=True)
            gidx = am + float(BK) * c
            better = mn < bv
            bi = jnp.where(better, gidx, bi)
            bv = jnp.where(better, mn, bv)
        idx_ref[...] = bi.astype(jnp.int32)

    return pl.pallas_call(
        body,
        grid=(NRB,),
        in_specs=([pl.BlockSpec((BM, IN), lambda i: (i, 0))]
                  + _const_specs(ws)
                  + [pl.BlockSpec((K, CD), lambda i: (0, 0))]),
        out_specs=[pl.BlockSpec((BM, CD), lambda i: (i, 0)),
                   pl.BlockSpec((BM, 1), lambda i: (i, 0))],
        out_shape=[jax.ShapeDtypeStruct((M, CD), _f32),
                   jax.ShapeDtypeStruct((M, 1), jnp.int32)],
        scratch_shapes=[pltpu.VMEM((K, CD), _bf16)],
    )(x2d, *ws, P["emb"])


def _dec_mega(zq, P, x2d, zn):
    ws = _block_weights(P, "dec_") + [
        P["dp1_W"], P["dp1_b"].reshape(1, ED),
        P["dp2_W"], P["dp2_b"].reshape(1, OUT)]

    def body(z_ref, *refs):
        x_ref, zn_ref, loss_ref = refs[-3], refs[-2], refs[-1]

        @pl.when(pl.program_id(0) == 0)
        def _init():
            loss_ref[...] = jnp.zeros((1, 1), _f32)

        zq_blk = z_ref[...]
        n = jnp.sqrt(jnp.sum(zq_blk * zq_blk, axis=1, keepdims=True))
        vn = zq_blk / jnp.maximum(n, 1e-12)
        dv = zn_ref[...] - vn
        vq_part = jnp.sum(dv * dv)

        h = _block_body(zq_blk, refs[:14], "dec_")
        dp1_W, dp1_b, dp2_W, dp2_b = [r[...] for r in refs[14:18]]
        t = jnp.tanh((_dot(h, dp1_W) + dp1_b).astype(_bf16))
        xr = _dot(t, dp2_W) + dp2_b
        d = xr - x_ref[...]
        rec_part = jnp.sum(d * d)
        loss_ref[...] += (rec_part * (1.0 / (M * IN))
                          + vq_part * (1.25 / (M * CD))).reshape(1, 1)

    return pl.pallas_call(
        body,
        grid=(NRB,),
        in_specs=([pl.BlockSpec((BM, CD), lambda i: (i, 0))]
                  + _const_specs(ws)
                  + [pl.BlockSpec((BM, OUT), lambda i: (i, 0)),
                     pl.BlockSpec((BM, CD), lambda i: (i, 0))]),
        out_specs=pl.BlockSpec((1, 1), lambda i: (0, 0)),
        out_shape=jax.ShapeDtypeStruct((1, 1), _f32),
    )(zq, *ws, x2d, zn)


def _sc_gather(table, idx):
    info = plsc.get_sparse_core_info()
    nw = info.num_cores * info.num_subcores
    b_per_w = M // nw
    mesh = plsc.VectorSubcoreMesh(core_axis_name="c", subcore_axis_name="s")

    @functools.partial(
        pl.kernel,
        mesh=mesh,
        out_type=jax.ShapeDtypeStruct((M, CD), _f32),
        scratch_types=[
            pltpu.VMEM((b_per_w,), jnp.int32),
            pltpu.VMEM((b_per_w, CD), _f32),
            pltpu.SemaphoreType.DMA,
        ],
    )
    def gather_kernel(table_hbm, idx_hbm, out_hbm, idx_v, rows_v, sem):
        wid = lax.axis_index("s") * info.num_cores + lax.axis_index("c")
        base = wid * b_per_w
        pltpu.sync_copy(idx_hbm.at[pl.ds(base, b_per_w)], idx_v)
        pltpu.async_copy(table_hbm.at[idx_v], rows_v, sem).wait()
        pltpu.sync_copy(rows_v, out_hbm.at[pl.ds(base, b_per_w)])

    return gather_kernel(table, idx)


def kernel(x, params):
    P = params
    x2d = x.reshape(M, IN)
    zn, idx = _enc_mega(x2d, P)
    zq = _sc_gather(P["emb"], idx.reshape(M))
    return _dec_mega(zq, P, x2d, zn)[0, 0]
